# baseline (speedup 1.0000x reference)
import functools

import jax
import jax.numpy as jnp
from jax import lax
from jax.experimental import pallas as pl
from jax.experimental.pallas import tpu as pltpu
from jax.experimental.pallas import tpu_sc as plsc

EMB = 64
NUM_MOD = 32
TILE = 512

N = 50000
H = 25000
NH = 25088
N2 = 2 * NH
PW = 25088
E2 = 32 * PW
CH = 1568
GP = CH // 16
EW = PW + 128
TGD = NH + H

_HI = jax.lax.Precision.HIGHEST


def _bfdot(a, w):
    return jnp.dot(a.astype(jnp.bfloat16), w.astype(jnp.bfloat16),
                   preferred_element_type=jnp.float32)

_mesh = plsc.VectorSubcoreMesh(core_axis_name="c", subcore_axis_name="s")


def _sc_partition(dstp, srcp):
    outs = [jax.ShapeDtypeStruct((2, 32, EW), jnp.int32),
            jax.ShapeDtypeStruct((2, 32, EW), jnp.int32),
            jax.ShapeDtypeStruct((2, 32, 16), jnp.int32)]

    @functools.partial(
        pl.kernel, mesh=_mesh, out_type=outs,
        compiler_params=pltpu.CompilerParams(needs_layout_passes=False, use_tc_tiling_on_sc=False),
        scratch_types=[
            pltpu.VMEM((CH,), jnp.int32),
            pltpu.VMEM((CH,), jnp.int32),
            pltpu.VMEM((EW,), jnp.int32),
            pltpu.VMEM((EW,), jnp.int32),
            pltpu.VMEM((EW,), jnp.int32),
            pltpu.VMEM((EW,), jnp.int32),
            pltpu.VMEM((16,), jnp.int32),
        ])
    def k(dst_h, src_h, cd_h, cs_h, cnt_h,
          dbuf, sbuf, cd0v, cs0v, cd1v, cs1v, cntv):
        c = lax.axis_index("c")
        s = lax.axis_index("s")
        w = s * 2 + c
        lanes = lax.iota(jnp.int32, 16)

        def chunk(kk, offs):
            pltpu.sync_copy(dst_h.at[pl.ds(w * PW + kk * CH, CH)], dbuf)
            pltpu.sync_copy(src_h.at[pl.ds(w * PW + kk * CH, CH)], sbuf)

            def grp(g, offs):
                off0, off1 = offs
                d = dbuf[pl.ds(g * 16, 16)]
                sv = sbuf[pl.ds(g * 16, 16)]
                m1 = d >= NH
                m0 = jnp.logical_not(m1)
                m0i = m0.astype(jnp.int32)
                cum = plsc.cumsum(m0i)
                exc = cum - m0i
                cnt0 = jnp.max(cum)
                pos0 = off0 + exc
                pos1 = off1 + lanes - exc
                plsc.store_scatter(cd0v, [pos0], d, mask=m0)
                plsc.store_scatter(cs0v, [pos0], sv, mask=m0)
                plsc.store_scatter(cd1v, [pos1], d, mask=m1)
                plsc.store_scatter(cs1v, [pos1], sv, mask=m1)
                return off0 + cnt0, off1 + (16 - cnt0)

            return lax.fori_loop(0, GP, grp, offs)

        off0, off1 = lax.fori_loop(0, PW // CH, chunk,
                                   (jnp.int32(0), jnp.int32(0)))

        tg = jnp.zeros((16,), jnp.int32) + TGD
        zz = jnp.zeros((16,), jnp.int32)
        for g in range(8):
            i0 = off0 + g * 16 + lanes
            i1 = off1 + g * 16 + lanes
            plsc.store_scatter(cd0v, [i0], tg)
            plsc.store_scatter(cs0v, [i0], zz)
            plsc.store_scatter(cd1v, [i1], tg)
            plsc.store_scatter(cs1v, [i1], zz)

        pltpu.sync_copy(cd0v, cd_h.at[0, w])
        pltpu.sync_copy(cs0v, cs_h.at[0, w])
        pltpu.sync_copy(cd1v, cd_h.at[1, w])
        pltpu.sync_copy(cs1v, cs_h.at[1, w])
        cntv[...] = zz + off0
        pltpu.sync_copy(cntv, cnt_h.at[0, w])
        cntv[...] = zz + off1
        pltpu.sync_copy(cntv, cnt_h.at[1, w])

    return k(dstp, srcp)


def _sc_edge(A, B, cd, cs, cnt, gbc, bbc):
    @functools.partial(
        pl.kernel, mesh=_mesh,
        out_type=jax.ShapeDtypeStruct((N2, EMB), jnp.float32),
        compiler_params=pltpu.CompilerParams(needs_layout_passes=False, use_tc_tiling_on_sc=False),
        scratch_types=[
            pltpu.VMEM((128,), jnp.int32),
            pltpu.VMEM((128,), jnp.int32),
            pltpu.VMEM((128,), jnp.int32),
            pltpu.VMEM((128, 64), jnp.float32),
            pltpu.VMEM((128, 64), jnp.float32),
            pltpu.VMEM((128, EMB), jnp.float32),
            pltpu.VMEM((64, 16), jnp.float32),
            pltpu.VMEM((64, 16), jnp.float32),
            pltpu.VMEM((64, 16), jnp.float32),
            pltpu.VMEM((16,), jnp.int32),
            pltpu.VMEM_SHARED((NH, EMB), jnp.float32),
            pltpu.SemaphoreType.DMA,
            pltpu.SemaphoreType.DMA,
        ])
    def k(A_h, B_h, cd_h, cs_h, cnt_h, g_h, b_h, out_h,
          dvm, svm, lidxv, arows, brows, rbuf, pre_t, gvm, bvm, cntv,
          acc, sema, semb):
        c = lax.axis_index("c")
        s = lax.axis_index("s")
        lanes = lax.iota(jnp.int32, 16)
        zf = jnp.zeros((16,), jnp.float32)

        pltpu.sync_copy(g_h, gvm)
        pltpu.sync_copy(b_h, bvm)

        def zr(r, _):
            for j in range(4):
                rbuf[r, pl.ds(j * 16, 16)] = zf
            return 0
        lax.fori_loop(0, 128, zr, 0)
        row0 = s * 1568
        for i in range(12):
            pltpu.sync_copy(rbuf, acc.at[pl.ds(row0 + i * 128, 128)])
        pltpu.sync_copy(rbuf.at[pl.ds(0, 32)], acc.at[pl.ds(row0 + 1536, 32)])
        plsc.subcore_barrier()

        def do_worker(w):
            pltpu.sync_copy(cnt_h.at[c, w], cntv)
            nk = (jnp.max(cntv[...]) + 127) // 128

            def chunk(kk, _):
                pltpu.sync_copy(cd_h.at[c, w, pl.ds(kk * 128, 128)], dvm)
                pltpu.sync_copy(cs_h.at[c, w, pl.ds(kk * 128, 128)], svm)

                def li(g, _):
                    d = dvm[pl.ds(g * 16, 16)]
                    lr = d - c * NH
                    ok = jnp.logical_and(lr >= 0, lr < H)
                    lidxv[pl.ds(g * 16, 16)] = jnp.where(ok, lr, H)
                    return 0
                lax.fori_loop(0, 8, li, 0)

                ca = pltpu.async_copy(A_h.at[dvm], arows, sema)
                cb = pltpu.async_copy(B_h.at[svm], brows, semb)
                ca.wait()
                cb.wait()

                def grp(g, _):
                    rv = g * 16 + lanes
                    sm = zf
                    sq = zf
                    for j in range(64):
                        jj = jnp.zeros((16,), jnp.int32) + j
                        a = plsc.load_gather(arows, [rv, jj])
                        b = plsc.load_gather(brows, [rv, jj])
                        p = a + b
                        pre_t[j] = p
                        sm = sm + p
                        sq = sq + p * p
                    mean = sm * (1.0 / 64.0)
                    var = sq * (1.0 / 64.0) - mean * mean + 1e-5
                    ih = plsc.bitcast(var, jnp.int32)
                    ih = 0x5F3759DF - (ih >> 1)
                    y = plsc.bitcast(ih, jnp.float32)
                    for _ in range(3):
                        y = y * (1.5 - 0.5 * var * y * y)
                    for j in range(64):
                        jj = jnp.zeros((16,), jnp.int32) + j
                        o = (pre_t[j] - mean) * y * gvm[j] + bvm[j]
                        o = jnp.maximum(o, 0.0)
                        oi = plsc.bitcast(o, jnp.int32)
                        oi = oi + 0x7FFF + ((oi >> 16) & 1)
                        oi = jnp.bitwise_and(oi, -65536)
                        o = plsc.bitcast(oi, jnp.float32)
                        plsc.store_scatter(rbuf, [rv, jj], o)
                    return 0
                lax.fori_loop(0, 8, grp, 0)
                pltpu.sync_copy(rbuf, acc.at[lidxv], add=True)
                return 0
            lax.fori_loop(0, nk, chunk, 0)

        do_worker(2 * s)
        do_worker(2 * s + 1)
        plsc.subcore_barrier()
        pltpu.sync_copy(acc.at[pl.ds(row0, 1568)],
                        out_h.at[pl.ds(c * NH + row0, 1568)])

    return k(A, B, cd, cs, cnt, gbc, bbc)


def _sc_deg(cd, cnt):
    @functools.partial(
        pl.kernel, mesh=_mesh,
        out_type=jax.ShapeDtypeStruct((N2, 16), jnp.float32),
        compiler_params=pltpu.CompilerParams(needs_layout_passes=False, use_tc_tiling_on_sc=False),
        scratch_types=[
            pltpu.VMEM((128,), jnp.int32),
            pltpu.VMEM((128,), jnp.int32),
            pltpu.VMEM((128, 16), jnp.float32),
            pltpu.VMEM((16,), jnp.int32),
            pltpu.VMEM_SHARED((NH, 16), jnp.float32),
        ])
    def k(cd_h, cnt_h, out_h, dvm, lidxv, obuf, cntv, dacc):
        c = lax.axis_index("c")
        s = lax.axis_index("s")
        zf = jnp.zeros((16,), jnp.float32)

        def zr(r, _):
            obuf[r] = zf
            return 0
        lax.fori_loop(0, 128, zr, 0)
        row0 = s * 1568
        for i in range(12):
            pltpu.sync_copy(obuf, dacc.at[pl.ds(row0 + i * 128, 128)])
        pltpu.sync_copy(obuf.at[pl.ds(0, 32)], dacc.at[pl.ds(row0 + 1536, 32)])

        def o1(r, _):
            obuf[r] = zf + 1.0
            return 0
        lax.fori_loop(0, 128, o1, 0)
        plsc.subcore_barrier()

        def do_worker(w):
            pltpu.sync_copy(cnt_h.at[c, w], cntv)
            nk = (jnp.max(cntv[...]) + 127) // 128

            def chunk(kk, _):
                pltpu.sync_copy(cd_h.at[c, w, pl.ds(kk * 128, 128)], dvm)

                def li(g, _):
                    d = dvm[pl.ds(g * 16, 16)]
                    lr = d - c * NH
                    ok = jnp.logical_and(lr >= 0, lr < H)
                    lidxv[pl.ds(g * 16, 16)] = jnp.where(ok, lr, H)
                    return 0
                lax.fori_loop(0, 8, li, 0)
                pltpu.sync_copy(obuf, dacc.at[lidxv], add=True)
                return 0
            lax.fori_loop(0, nk, chunk, 0)

        do_worker(2 * s)
        do_worker(2 * s + 1)
        plsc.subcore_barrier()
        pltpu.sync_copy(dacc.at[pl.ds(row0, 1568)],
                        out_h.at[pl.ds(c * NH + row0, 1568)])

    return k(cd, cnt)


def _ln_rows(x, g, b, eps=1e-5):
    m = jnp.mean(x, axis=-1, keepdims=True)
    v = jnp.mean((x - m) ** 2, axis=-1, keepdims=True)
    return (x - m) / jnp.sqrt(v + eps) * g + b


def _emb2_kernel(x_ref, w1_ref, b1_ref, w2_ref, b2_ref, g_ref, bb_ref, o_ref):
    x = x_ref[...]
    h = _ln_rows(x, g_ref[...], bb_ref[...])
    h = jax.nn.relu(_bfdot(h, w1_ref[...]) + b1_ref[...])
    o_ref[...] = jax.nn.relu(_bfdot(h, w2_ref[...]) + b2_ref[...])


def _emb2(x, p):
    F = x.shape[1]
    grid = N2 // TILE
    return pl.pallas_call(
        _emb2_kernel,
        grid=(grid,),
        in_specs=[
            pl.BlockSpec((TILE, F), lambda i: (i, 0)),
            pl.BlockSpec((F, EMB), lambda i: (0, 0)),
            pl.BlockSpec((EMB,), lambda i: (0,)),
            pl.BlockSpec((EMB, EMB), lambda i: (0, 0)),
            pl.BlockSpec((EMB,), lambda i: (0,)),
            pl.BlockSpec((F,), lambda i: (0,)),
            pl.BlockSpec((F,), lambda i: (0,)),
        ],
        out_specs=pl.BlockSpec((TILE, EMB), lambda i: (i, 0)),
        out_shape=jax.ShapeDtypeStruct((N2, EMB), jnp.float32),
    )(x, p['W1'], p['b1'], p['W2'], p['b2'], p['ln_g'], p['ln_b'])


def _ab_kernel(r_ref, l_ref, wl_ref, bl_ref, wr_ref, a_ref, b_ref):
    a_ref[...] = _bfdot(r_ref[...], wl_ref[...]) + bl_ref[...]
    b_ref[...] = _bfdot(l_ref[...], wr_ref[...])


def _ab(right, left, wl, bl, wr):
    grid = N2 // TILE
    return pl.pallas_call(
        _ab_kernel,
        grid=(grid,),
        in_specs=[
            pl.BlockSpec((TILE, EMB), lambda i: (i, 0)),
            pl.BlockSpec((TILE, EMB), lambda i: (i, 0)),
            pl.BlockSpec((EMB, EMB), lambda i: (0, 0)),
            pl.BlockSpec((EMB,), lambda i: (0,)),
            pl.BlockSpec((EMB, EMB), lambda i: (0, 0)),
        ],
        out_specs=[
            pl.BlockSpec((TILE, EMB), lambda i: (i, 0)),
            pl.BlockSpec((TILE, EMB), lambda i: (i, 0)),
        ],
        out_shape=[
            jax.ShapeDtypeStruct((N2, EMB), jnp.float32),
            jax.ShapeDtypeStruct((N2, EMB), jnp.float32),
        ],
    )(right, left, wl, bl, wr)


def _agg_cat(sd, deg, rt, fw, fb, pg, pb):
    agg = (jnp.dot(sd, fw, preferred_element_type=jnp.float32,
                   precision=_HI) + deg[:, :1] * fb)
    aggl = _ln_rows(agg, pg, pb)
    return jnp.concatenate([aggl, rt], axis=-1)


def _post_plain_kernel(sd_ref, deg_ref, rt_ref, fw_ref, fb_ref, pg_ref, pb_ref,
                       w1_ref, b1_ref, w2_ref, b2_ref, o_ref):
    cat = _agg_cat(sd_ref[...], deg_ref[...], rt_ref[...], fw_ref[...],
                   fb_ref[...], pg_ref[...], pb_ref[...])
    h = jax.nn.relu(_bfdot(cat, w1_ref[...]) + b1_ref[...])
    o_ref[...] = _bfdot(h, w2_ref[...]) + b2_ref[...]


def _post_plain(SD, deg, right, p):
    grid = N2 // TILE
    return pl.pallas_call(
        _post_plain_kernel,
        grid=(grid,),
        in_specs=[
            pl.BlockSpec((TILE, EMB), lambda i: (i, 0)),
            pl.BlockSpec((TILE, 16), lambda i: (i, 0)),
            pl.BlockSpec((TILE, EMB), lambda i: (i, 0)),
            pl.BlockSpec((EMB, EMB), lambda i: (0, 0)),
            pl.BlockSpec((EMB,), lambda i: (0,)),
            pl.BlockSpec((EMB,), lambda i: (0,)),
            pl.BlockSpec((EMB,), lambda i: (0,)),
            pl.BlockSpec((2 * EMB, EMB), lambda i: (0, 0)),
            pl.BlockSpec((EMB,), lambda i: (0,)),
            pl.BlockSpec((EMB, EMB), lambda i: (0, 0)),
            pl.BlockSpec((EMB,), lambda i: (0,)),
        ],
        out_specs=pl.BlockSpec((TILE, EMB), lambda i: (i, 0)),
        out_shape=jax.ShapeDtypeStruct((N2, EMB), jnp.float32),
    )(SD, deg, right, p['fmf_W'], p['fmf_b'], p['post_ln_g'], p['post_ln_b'],
      p['out_W1'], p['out_b1'], p['out_W2'], p['out_b2'])


def _post_color_kernel(sd_ref, deg_ref, rt_ref, col_ref, fw_ref, fb_ref,
                       pg_ref, pb_ref, w1_ref, b1_ref, w2_ref, b2_ref, o_ref):
    cat = _agg_cat(sd_ref[...], deg_ref[...], rt_ref[...], fw_ref[...],
                   fb_ref[...], pg_ref[...], pb_ref[...])
    col = col_ref[...]
    acc = jnp.zeros((cat.shape[0], EMB), jnp.float32)
    catb = cat.astype(jnp.bfloat16)
    for c in range(NUM_MOD):
        h = jax.nn.relu(jnp.dot(catb, w1_ref[c].astype(jnp.bfloat16),
                                preferred_element_type=jnp.float32) + b1_ref[c])
        z = _bfdot(h, w2_ref[c]) + b2_ref[c]
        acc = jnp.where(col == c, z, acc)
    o_ref[...] = acc


def _post_color(SD, deg, right, color, p, cp):
    grid = N2 // TILE
    return pl.pallas_call(
        _post_color_kernel,
        grid=(grid,),
        in_specs=[
            pl.BlockSpec((TILE, EMB), lambda i: (i, 0)),
            pl.BlockSpec((TILE, 16), lambda i: (i, 0)),
            pl.BlockSpec((TILE, EMB), lambda i: (i, 0)),
            pl.BlockSpec((TILE, 1), lambda i: (i, 0)),
            pl.BlockSpec((EMB, EMB), lambda i: (0, 0)),
            pl.BlockSpec((EMB,), lambda i: (0,)),
            pl.BlockSpec((EMB,), lambda i: (0,)),
            pl.BlockSpec((EMB,), lambda i: (0,)),
            pl.BlockSpec((NUM_MOD, 2 * EMB, EMB), lambda i: (0, 0, 0)),
            pl.BlockSpec((NUM_MOD, EMB), lambda i: (0, 0)),
            pl.BlockSpec((NUM_MOD, EMB, EMB), lambda i: (0, 0, 0)),
            pl.BlockSpec((NUM_MOD, EMB), lambda i: (0, 0)),
        ],
        out_specs=pl.BlockSpec((TILE, EMB), lambda i: (i, 0)),
        out_shape=jax.ShapeDtypeStruct((N2, EMB), jnp.float32),
    )(SD, deg, right, color, p['fmf_W'], p['fmf_b'], p['post_ln_g'],
      p['post_ln_b'], cp['W1'], cp['b1'], cp['W2'], cp['b2'])


def _head_kernel(x_ref, w1_ref, b1_ref, w2_ref, o_ref):
    h = jax.nn.relu(_bfdot(x_ref[...], w1_ref[...]) + b1_ref[...])
    o_ref[...] = _bfdot(h, w2_ref[...])


def _head(x, w1, b1, w2):
    grid = N2 // TILE
    w2p = jnp.pad(w2, ((0, 0), (0, 127)))
    return pl.pallas_call(
        _head_kernel,
        grid=(grid,),
        in_specs=[
            pl.BlockSpec((TILE, EMB), lambda i: (i, 0)),
            pl.BlockSpec((EMB, EMB), lambda i: (0, 0)),
            pl.BlockSpec((EMB,), lambda i: (0,)),
            pl.BlockSpec((EMB, 128), lambda i: (0, 0)),
        ],
        out_specs=pl.BlockSpec((TILE, 128), lambda i: (i, 0)),
        out_shape=jax.ShapeDtypeStruct((N2, 128), jnp.float32),
    )(x, w1, b1, w2p)


def _to_layout(x):
    z = jnp.zeros((NH - H,) + x.shape[1:], x.dtype)
    return jnp.concatenate([x[:H], z, x[H:], z], axis=0)


def kernel(constraint_features, edge_indices, edge_features, variable_features,
           variableColor, consColor, params):
    cons = _emb2(_to_layout(constraint_features), params['cons_emb'])
    var = _emb2(_to_layout(variable_features), params['var_emb'])

    c_vec = params['edge_ln']['b'][0]

    ccol = _to_layout(consColor)[:, None]
    vcol = _to_layout(variableColor)[:, None]

    ec = edge_indices[0]
    ev = edge_indices[1]
    ecp = ec + (ec >= H).astype(jnp.int32) * (NH - H)
    evp = ev + (ev >= H).astype(jnp.int32) * (NH - H)
    npad_e = E2 - ec.shape[0]
    padd = jnp.full((npad_e,), TGD, jnp.int32)
    padz = jnp.zeros((npad_e,), jnp.int32)

    cd_vc, cs_vc, cnt_vc = _sc_partition(jnp.concatenate([ecp, padd]),
                                         jnp.concatenate([evp, padz]))
    cd_cv, cs_cv, cnt_cv = _sc_partition(jnp.concatenate([evp, padd]),
                                         jnp.concatenate([ecp, padz]))
    deg_vc = _sc_deg(cd_vc, cnt_vc)
    deg_cv = _sc_deg(cd_cv, cnt_cv)

    def conv(left, right, part, p, color=None, cp=None):
        cd, cs, cnt, deg = part
        bias = p['fml_b'] + c_vec * p['fme_W'][0]
        A, B = _ab(right, left, p['fml_W'], bias, p['fmr_W'])
        p = dict(p)
        p['fmf_W'] = p['fmf_W'].astype(jnp.bfloat16).astype(jnp.float32)
        gbc = jnp.broadcast_to(p['fmf_ln_g'][:, None], (EMB, 16))
        bbc = jnp.broadcast_to(p['fmf_ln_b'][:, None], (EMB, 16))
        SD = _sc_edge(A, B, cd, cs, cnt, gbc, bbc)
        if color is None:
            return _post_plain(SD, deg, right, p)
        return _post_color(SD, deg, right, color, p, cp)

    cp = params['color']
    part_vc = (cd_vc, cs_vc, cnt_vc, deg_vc)
    part_cv = (cd_cv, cs_cv, cnt_cv, deg_cv)
    cons = conv(var, cons, part_vc, params['conv_v_to_c'], ccol, cp)
    var = conv(cons, var, part_cv, params['conv_c_to_v'], vcol, cp)
    cons = conv(var, cons, part_vc, params['conv_v_to_c2'])
    var = conv(cons, var, part_cv, params['conv_c_to_v2'])

    out = _head(var, params['out']['W1'], params['out']['b1'],
                params['out']['W2'])
    return jnp.concatenate([out[:H, 0], out[NH:NH + H, 0]])


# speedup vs baseline: 1.0450x; 1.0450x over previous
import functools

import jax
import jax.numpy as jnp
from jax import lax
from jax.experimental import pallas as pl
from jax.experimental.pallas import tpu as pltpu
from jax.experimental.pallas import tpu_sc as plsc

EMB = 64
NUM_MOD = 32
TILE = 512

N = 50000
H = 25000
NH = 25088
N2 = 2 * NH
PW = 25088
E2 = 32 * PW
CH = 1568
GP = CH // 16
EW = PW + 128
TGD = NH + H
CHK = 48

_HI = jax.lax.Precision.HIGHEST


def _bfdot(a, w):
    return jnp.dot(a.astype(jnp.bfloat16), w.astype(jnp.bfloat16),
                   preferred_element_type=jnp.float32)

_mesh = plsc.VectorSubcoreMesh(core_axis_name="c", subcore_axis_name="s")


def _sc_partition(dstp, srcp):
    outs = [jax.ShapeDtypeStruct((2, 32, EW), jnp.int32),
            jax.ShapeDtypeStruct((2, 32, EW), jnp.int32),
            jax.ShapeDtypeStruct((2, 32, 16), jnp.int32)]

    @functools.partial(
        pl.kernel, mesh=_mesh, out_type=outs,
        compiler_params=pltpu.CompilerParams(needs_layout_passes=False, use_tc_tiling_on_sc=False),
        scratch_types=[
            pltpu.VMEM((CH,), jnp.int32),
            pltpu.VMEM((CH,), jnp.int32),
            pltpu.VMEM((EW,), jnp.int32),
            pltpu.VMEM((EW,), jnp.int32),
            pltpu.VMEM((EW,), jnp.int32),
            pltpu.VMEM((EW,), jnp.int32),
            pltpu.VMEM((16,), jnp.int32),
        ])
    def k(dst_h, src_h, cd_h, cs_h, cnt_h,
          dbuf, sbuf, cd0v, cs0v, cd1v, cs1v, cntv):
        c = lax.axis_index("c")
        s = lax.axis_index("s")
        w = s * 2 + c
        lanes = lax.iota(jnp.int32, 16)

        def chunk(kk, offs):
            pltpu.sync_copy(dst_h.at[pl.ds(w * PW + kk * CH, CH)], dbuf)
            pltpu.sync_copy(src_h.at[pl.ds(w * PW + kk * CH, CH)], sbuf)

            def grp(g, offs):
                off0, off1 = offs
                d = dbuf[pl.ds(g * 16, 16)]
                sv = sbuf[pl.ds(g * 16, 16)]
                m1 = d >= NH
                m0 = jnp.logical_not(m1)
                m0i = m0.astype(jnp.int32)
                cum = plsc.cumsum(m0i)
                exc = cum - m0i
                cnt0 = jnp.max(cum)
                pos0 = off0 + exc
                pos1 = off1 + lanes - exc
                plsc.store_scatter(cd0v, [pos0], d, mask=m0)
                plsc.store_scatter(cs0v, [pos0], sv, mask=m0)
                plsc.store_scatter(cd1v, [pos1], d, mask=m1)
                plsc.store_scatter(cs1v, [pos1], sv, mask=m1)
                return off0 + cnt0, off1 + (16 - cnt0)

            return lax.fori_loop(0, GP, grp, offs)

        off0, off1 = lax.fori_loop(0, PW // CH, chunk,
                                   (jnp.int32(0), jnp.int32(0)))

        tg = jnp.zeros((16,), jnp.int32) + TGD
        zz = jnp.zeros((16,), jnp.int32)
        for g in range(16):
            i0 = off0 + g * 16 + lanes
            i1 = off1 + g * 16 + lanes
            plsc.store_scatter(cd0v, [i0], tg)
            plsc.store_scatter(cs0v, [i0], zz)
            plsc.store_scatter(cd1v, [i1], tg)
            plsc.store_scatter(cs1v, [i1], zz)

        pltpu.sync_copy(cd0v, cd_h.at[0, w])
        pltpu.sync_copy(cs0v, cs_h.at[0, w])
        pltpu.sync_copy(cd1v, cd_h.at[1, w])
        pltpu.sync_copy(cs1v, cs_h.at[1, w])
        cntv[...] = zz + off0
        pltpu.sync_copy(cntv, cnt_h.at[0, w])
        cntv[...] = zz + off1
        pltpu.sync_copy(cntv, cnt_h.at[1, w])

    return k(dstp, srcp)


def _sc_edge(A, B, cd, cs, cnt, gbc, bbc):
    @functools.partial(
        pl.kernel, mesh=_mesh,
        out_type=jax.ShapeDtypeStruct((N2, EMB), jnp.float32),
        compiler_params=pltpu.CompilerParams(needs_layout_passes=False, use_tc_tiling_on_sc=False),
        scratch_types=[
            [pltpu.VMEM((CHK,), jnp.int32)] * 2,
            [pltpu.VMEM((CHK,), jnp.int32)] * 2,
            [pltpu.VMEM((CHK,), jnp.int32)] * 2,
            [pltpu.VMEM((CHK, EMB), jnp.float32)] * 2,
            [pltpu.VMEM((CHK, EMB), jnp.float32)] * 2,
            [pltpu.VMEM((CHK, EMB), jnp.float32)] * 2,
            pltpu.VMEM((EMB, 16), jnp.float32),
            pltpu.VMEM((EMB, 16), jnp.float32),
            pltpu.VMEM((EMB, 16), jnp.float32),
            pltpu.VMEM((16,), jnp.int32),
            pltpu.VMEM_SHARED((NH, EMB), jnp.float32),
            [pltpu.SemaphoreType.DMA] * 2,
            [pltpu.SemaphoreType.DMA] * 2,
            [pltpu.SemaphoreType.DMA] * 2,
        ])
    def k(A_h, B_h, cd_h, cs_h, cnt_h, g_h, b_h, out_h,
          dsl, ssl, lidx, arows, brows, rbuf, pre_t, gvm, bvm, cntv,
          acc, semi, semg, semsc):
        c = lax.axis_index("c")
        s = lax.axis_index("s")
        lanes = lax.iota(jnp.int32, 16)
        zf = jnp.zeros((16,), jnp.float32)

        pltpu.sync_copy(g_h, gvm)
        pltpu.sync_copy(b_h, bvm)

        def zr(r, _):
            for j in range(4):
                rbuf[0][r, pl.ds(j * 16, 16)] = zf
                rbuf[1][r, pl.ds(j * 16, 16)] = zf
            return 0
        lax.fori_loop(0, CHK, zr, 0)
        row0 = s * 1568
        for i in range(32):
            pltpu.sync_copy(rbuf[i % 2], acc.at[pl.ds(row0 + i * CHK, CHK)])
        pltpu.sync_copy(rbuf[0].at[pl.ds(0, 32)], acc.at[pl.ds(row0 + 1536, 32)])
        plsc.subcore_barrier()

        def idx_start(w, kk, b):
            pltpu.make_async_copy(cd_h.at[c, w, pl.ds(kk * CHK, CHK)],
                                  dsl[b], semi[b]).start()
            pltpu.make_async_copy(cs_h.at[c, w, pl.ds(kk * CHK, CHK)],
                                  ssl[b], semi[b]).start()

        def idx_wait(w, kk, b):
            pltpu.make_async_copy(cd_h.at[c, w, pl.ds(kk * CHK, CHK)],
                                  dsl[b], semi[b]).wait()
            pltpu.make_async_copy(cs_h.at[c, w, pl.ds(kk * CHK, CHK)],
                                  ssl[b], semi[b]).wait()

        def gather_start(b):
            pltpu.make_async_copy(A_h.at[dsl[b]], arows[b], semg[b]).start()
            pltpu.make_async_copy(B_h.at[ssl[b]], brows[b], semg[b]).start()

        def gather_wait(b):
            pltpu.make_async_copy(A_h.at[dsl[b]], arows[b], semg[b]).wait()
            pltpu.make_async_copy(B_h.at[ssl[b]], brows[b], semg[b]).wait()

        def scat_start(b):
            pltpu.make_async_copy(rbuf[b], acc.at[lidx[b]], semsc[b]).start(add=True)

        def scat_wait(b):
            pltpu.make_async_copy(rbuf[b], acc.at[lidx[b]], semsc[b]).wait()

        def compute_chunk(b):
            def grp(g, _):
                rv = g * 16 + lanes
                sm = zf
                sq = zf
                for j in range(EMB):
                    jj = jnp.zeros((16,), jnp.int32) + j
                    a = plsc.load_gather(arows[b], [rv, jj])
                    bb = plsc.load_gather(brows[b], [rv, jj])
                    p = a + bb
                    pre_t[j] = p
                    sm = sm + p
                    sq = sq + p * p
                mean = sm * (1.0 / EMB)
                var = sq * (1.0 / EMB) - mean * mean + 1e-5
                ih = plsc.bitcast(var, jnp.int32)
                ih = 0x5F3759DF - (ih >> 1)
                y = plsc.bitcast(ih, jnp.float32)
                for _ in range(3):
                    y = y * (1.5 - 0.5 * var * y * y)
                for j in range(EMB):
                    jj = jnp.zeros((16,), jnp.int32) + j
                    o = (pre_t[j] - mean) * y * gvm[j] + bvm[j]
                    o = jnp.maximum(o, 0.0)
                    oi = plsc.bitcast(o, jnp.int32)
                    oi = oi + 0x7FFF + ((oi >> 16) & 1)
                    oi = jnp.bitwise_and(oi, -65536)
                    o = plsc.bitcast(oi, jnp.float32)
                    plsc.store_scatter(rbuf[b], [rv, jj], o)
                return 0
            lax.fori_loop(0, CHK // 16, grp, 0)

        def do_worker(w):
            pltpu.sync_copy(cnt_h.at[c, w], cntv)
            cntw = jnp.max(cntv[...])
            nk = ((cntw + 2 * CHK - 1) // (2 * CHK)) * 2

            @pl.when(nk > 0)
            def _prime():
                pltpu.sync_copy(cd_h.at[c, w, pl.ds(0, CHK)], dsl[0])
                pltpu.sync_copy(cs_h.at[c, w, pl.ds(0, CHK)], ssl[0])
                gather_start(0)
                idx_start(w, 1, 1)

            def pair(jj, _):
                for b in range(2):
                    kk = 2 * jj + b
                    gather_wait(b)

                    @pl.when(kk >= 2)
                    def _():
                        scat_wait(b)
                    compute_lidx_and_rows(w, kk, b)
                return 0

            def compute_lidx_and_rows(w, kk, b):
                def li(g, _):
                    d = dsl[b][pl.ds(g * 16, 16)]
                    lr = d - c * NH
                    ok = jnp.logical_and(lr >= 0, lr < H)
                    lidx[b][pl.ds(g * 16, 16)] = jnp.where(ok, lr, H)
                    return 0
                lax.fori_loop(0, CHK // 16, li, 0)

                @pl.when(kk + 2 < nk)
                def _():
                    idx_start(w, kk + 2, b)

                @pl.when(kk + 1 < nk)
                def _():
                    idx_wait(w, kk + 1, 1 - b)
                    gather_start(1 - b)

                compute_chunk(b)
                scat_start(b)

            lax.fori_loop(0, nk // 2, pair, 0)

            @pl.when(nk >= 2)
            def _drain():
                scat_wait(0)
                scat_wait(1)

        do_worker(2 * s)
        do_worker(2 * s + 1)
        plsc.subcore_barrier()
        pltpu.sync_copy(acc.at[pl.ds(row0, 1568)],
                        out_h.at[pl.ds(c * NH + row0, 1568)])

    return k(A, B, cd, cs, cnt, gbc, bbc)


def _sc_deg(cd, cnt):
    @functools.partial(
        pl.kernel, mesh=_mesh,
        out_type=jax.ShapeDtypeStruct((N2, 16), jnp.float32),
        compiler_params=pltpu.CompilerParams(needs_layout_passes=False, use_tc_tiling_on_sc=False),
        scratch_types=[
            pltpu.VMEM((128,), jnp.int32),
            pltpu.VMEM((128,), jnp.int32),
            pltpu.VMEM((128, 16), jnp.float32),
            pltpu.VMEM((16,), jnp.int32),
            pltpu.VMEM_SHARED((NH, 16), jnp.float32),
        ])
    def k(cd_h, cnt_h, out_h, dvm, lidxv, obuf, cntv, dacc):
        c = lax.axis_index("c")
        s = lax.axis_index("s")
        zf = jnp.zeros((16,), jnp.float32)

        def zr(r, _):
            obuf[r] = zf
            return 0
        lax.fori_loop(0, 128, zr, 0)
        row0 = s * 1568
        for i in range(12):
            pltpu.sync_copy(obuf, dacc.at[pl.ds(row0 + i * 128, 128)])
        pltpu.sync_copy(obuf.at[pl.ds(0, 32)], dacc.at[pl.ds(row0 + 1536, 32)])

        def o1(r, _):
            obuf[r] = zf + 1.0
            return 0
        lax.fori_loop(0, 128, o1, 0)
        plsc.subcore_barrier()

        def do_worker(w):
            pltpu.sync_copy(cnt_h.at[c, w], cntv)
            nk = (jnp.max(cntv[...]) + 127) // 128

            def chunk(kk, _):
                pltpu.sync_copy(cd_h.at[c, w, pl.ds(kk * 128, 128)], dvm)

                def li(g, _):
                    d = dvm[pl.ds(g * 16, 16)]
                    lr = d - c * NH
                    ok = jnp.logical_and(lr >= 0, lr < H)
                    lidxv[pl.ds(g * 16, 16)] = jnp.where(ok, lr, H)
                    return 0
                lax.fori_loop(0, 8, li, 0)
                pltpu.sync_copy(obuf, dacc.at[lidxv], add=True)
                return 0
            lax.fori_loop(0, nk, chunk, 0)

        do_worker(2 * s)
        do_worker(2 * s + 1)
        plsc.subcore_barrier()
        pltpu.sync_copy(dacc.at[pl.ds(row0, 1568)],
                        out_h.at[pl.ds(c * NH + row0, 1568)])

    return k(cd, cnt)


def _ln_rows(x, g, b, eps=1e-5):
    m = jnp.mean(x, axis=-1, keepdims=True)
    v = jnp.mean((x - m) ** 2, axis=-1, keepdims=True)
    return (x - m) / jnp.sqrt(v + eps) * g + b


def _emb2_kernel(x_ref, w1_ref, b1_ref, w2_ref, b2_ref, g_ref, bb_ref, o_ref):
    x = x_ref[...]
    h = _ln_rows(x, g_ref[...], bb_ref[...])
    h = jax.nn.relu(_bfdot(h, w1_ref[...]) + b1_ref[...])
    o_ref[...] = jax.nn.relu(_bfdot(h, w2_ref[...]) + b2_ref[...])


def _emb2(x, p):
    F = x.shape[1]
    grid = N2 // TILE
    return pl.pallas_call(
        _emb2_kernel,
        grid=(grid,),
        in_specs=[
            pl.BlockSpec((TILE, F), lambda i: (i, 0)),
            pl.BlockSpec((F, EMB), lambda i: (0, 0)),
            pl.BlockSpec((EMB,), lambda i: (0,)),
            pl.BlockSpec((EMB, EMB), lambda i: (0, 0)),
            pl.BlockSpec((EMB,), lambda i: (0,)),
            pl.BlockSpec((F,), lambda i: (0,)),
            pl.BlockSpec((F,), lambda i: (0,)),
        ],
        out_specs=pl.BlockSpec((TILE, EMB), lambda i: (i, 0)),
        out_shape=jax.ShapeDtypeStruct((N2, EMB), jnp.float32),
    )(x, p['W1'], p['b1'], p['W2'], p['b2'], p['ln_g'], p['ln_b'])


def _ab_kernel(r_ref, l_ref, wl_ref, bl_ref, wr_ref, a_ref, b_ref):
    a_ref[...] = _bfdot(r_ref[...], wl_ref[...]) + bl_ref[...]
    b_ref[...] = _bfdot(l_ref[...], wr_ref[...])


def _ab(right, left, wl, bl, wr):
    grid = N2 // TILE
    return pl.pallas_call(
        _ab_kernel,
        grid=(grid,),
        in_specs=[
            pl.BlockSpec((TILE, EMB), lambda i: (i, 0)),
            pl.BlockSpec((TILE, EMB), lambda i: (i, 0)),
            pl.BlockSpec((EMB, EMB), lambda i: (0, 0)),
            pl.BlockSpec((EMB,), lambda i: (0,)),
            pl.BlockSpec((EMB, EMB), lambda i: (0, 0)),
        ],
        out_specs=[
            pl.BlockSpec((TILE, EMB), lambda i: (i, 0)),
            pl.BlockSpec((TILE, EMB), lambda i: (i, 0)),
        ],
        out_shape=[
            jax.ShapeDtypeStruct((N2, EMB), jnp.float32),
            jax.ShapeDtypeStruct((N2, EMB), jnp.float32),
        ],
    )(right, left, wl, bl, wr)


def _agg_cat(sd, deg, rt, fw, fb, pg, pb):
    agg = (jnp.dot(sd, fw, preferred_element_type=jnp.float32,
                   precision=_HI) + deg[:, :1] * fb)
    aggl = _ln_rows(agg, pg, pb)
    return jnp.concatenate([aggl, rt], axis=-1)


def _post_plain_kernel(sd_ref, deg_ref, rt_ref, fw_ref, fb_ref, pg_ref, pb_ref,
                       w1_ref, b1_ref, w2_ref, b2_ref, o_ref):
    cat = _agg_cat(sd_ref[...], deg_ref[...], rt_ref[...], fw_ref[...],
                   fb_ref[...], pg_ref[...], pb_ref[...])
    h = jax.nn.relu(_bfdot(cat, w1_ref[...]) + b1_ref[...])
    o_ref[...] = _bfdot(h, w2_ref[...]) + b2_ref[...]


def _post_plain(SD, deg, right, p):
    grid = N2 // TILE
    return pl.pallas_call(
        _post_plain_kernel,
        grid=(grid,),
        in_specs=[
            pl.BlockSpec((TILE, EMB), lambda i: (i, 0)),
            pl.BlockSpec((TILE, 16), lambda i: (i, 0)),
            pl.BlockSpec((TILE, EMB), lambda i: (i, 0)),
            pl.BlockSpec((EMB, EMB), lambda i: (0, 0)),
            pl.BlockSpec((EMB,), lambda i: (0,)),
            pl.BlockSpec((EMB,), lambda i: (0,)),
            pl.BlockSpec((EMB,), lambda i: (0,)),
            pl.BlockSpec((2 * EMB, EMB), lambda i: (0, 0)),
            pl.BlockSpec((EMB,), lambda i: (0,)),
            pl.BlockSpec((EMB, EMB), lambda i: (0, 0)),
            pl.BlockSpec((EMB,), lambda i: (0,)),
        ],
        out_specs=pl.BlockSpec((TILE, EMB), lambda i: (i, 0)),
        out_shape=jax.ShapeDtypeStruct((N2, EMB), jnp.float32),
    )(SD, deg, right, p['fmf_W'], p['fmf_b'], p['post_ln_g'], p['post_ln_b'],
      p['out_W1'], p['out_b1'], p['out_W2'], p['out_b2'])


def _post_color_kernel(sd_ref, deg_ref, rt_ref, col_ref, fw_ref, fb_ref,
                       pg_ref, pb_ref, w1_ref, b1_ref, w2_ref, b2_ref, o_ref):
    cat = _agg_cat(sd_ref[...], deg_ref[...], rt_ref[...], fw_ref[...],
                   fb_ref[...], pg_ref[...], pb_ref[...])
    col = col_ref[...]
    acc = jnp.zeros((cat.shape[0], EMB), jnp.float32)
    catb = cat.astype(jnp.bfloat16)
    for c in range(NUM_MOD):
        h = jax.nn.relu(jnp.dot(catb, w1_ref[c].astype(jnp.bfloat16),
                                preferred_element_type=jnp.float32) + b1_ref[c])
        z = _bfdot(h, w2_ref[c]) + b2_ref[c]
        acc = jnp.where(col == c, z, acc)
    o_ref[...] = acc


def _post_color(SD, deg, right, color, p, cp):
    grid = N2 // TILE
    return pl.pallas_call(
        _post_color_kernel,
        grid=(grid,),
        in_specs=[
            pl.BlockSpec((TILE, EMB), lambda i: (i, 0)),
            pl.BlockSpec((TILE, 16), lambda i: (i, 0)),
            pl.BlockSpec((TILE, EMB), lambda i: (i, 0)),
            pl.BlockSpec((TILE, 1), lambda i: (i, 0)),
            pl.BlockSpec((EMB, EMB), lambda i: (0, 0)),
            pl.BlockSpec((EMB,), lambda i: (0,)),
            pl.BlockSpec((EMB,), lambda i: (0,)),
            pl.BlockSpec((EMB,), lambda i: (0,)),
            pl.BlockSpec((NUM_MOD, 2 * EMB, EMB), lambda i: (0, 0, 0)),
            pl.BlockSpec((NUM_MOD, EMB), lambda i: (0, 0)),
            pl.BlockSpec((NUM_MOD, EMB, EMB), lambda i: (0, 0, 0)),
            pl.BlockSpec((NUM_MOD, EMB), lambda i: (0, 0)),
        ],
        out_specs=pl.BlockSpec((TILE, EMB), lambda i: (i, 0)),
        out_shape=jax.ShapeDtypeStruct((N2, EMB), jnp.float32),
    )(SD, deg, right, color, p['fmf_W'], p['fmf_b'], p['post_ln_g'],
      p['post_ln_b'], cp['W1'], cp['b1'], cp['W2'], cp['b2'])


def _head_kernel(x_ref, w1_ref, b1_ref, w2_ref, o_ref):
    h = jax.nn.relu(_bfdot(x_ref[...], w1_ref[...]) + b1_ref[...])
    o_ref[...] = _bfdot(h, w2_ref[...])


def _head(x, w1, b1, w2):
    grid = N2 // TILE
    w2p = jnp.pad(w2, ((0, 0), (0, 127)))
    return pl.pallas_call(
        _head_kernel,
        grid=(grid,),
        in_specs=[
            pl.BlockSpec((TILE, EMB), lambda i: (i, 0)),
            pl.BlockSpec((EMB, EMB), lambda i: (0, 0)),
            pl.BlockSpec((EMB,), lambda i: (0,)),
            pl.BlockSpec((EMB, 128), lambda i: (0, 0)),
        ],
        out_specs=pl.BlockSpec((TILE, 128), lambda i: (i, 0)),
        out_shape=jax.ShapeDtypeStruct((N2, 128), jnp.float32),
    )(x, w1, b1, w2p)


def _to_layout(x):
    z = jnp.zeros((NH - H,) + x.shape[1:], x.dtype)
    return jnp.concatenate([x[:H], z, x[H:], z], axis=0)


def kernel(constraint_features, edge_indices, edge_features, variable_features,
           variableColor, consColor, params):
    cons = _emb2(_to_layout(constraint_features), params['cons_emb'])
    var = _emb2(_to_layout(variable_features), params['var_emb'])

    c_vec = params['edge_ln']['b'][0]

    ccol = _to_layout(consColor)[:, None]
    vcol = _to_layout(variableColor)[:, None]

    ec = edge_indices[0]
    ev = edge_indices[1]
    ecp = ec + (ec >= H).astype(jnp.int32) * (NH - H)
    evp = ev + (ev >= H).astype(jnp.int32) * (NH - H)
    npad_e = E2 - ec.shape[0]
    padd = jnp.full((npad_e,), TGD, jnp.int32)
    padz = jnp.zeros((npad_e,), jnp.int32)

    cd_vc, cs_vc, cnt_vc = _sc_partition(jnp.concatenate([ecp, padd]),
                                         jnp.concatenate([evp, padz]))
    cd_cv, cs_cv, cnt_cv = _sc_partition(jnp.concatenate([evp, padd]),
                                         jnp.concatenate([ecp, padz]))
    deg_vc = _sc_deg(cd_vc, cnt_vc)
    deg_cv = _sc_deg(cd_cv, cnt_cv)

    def conv(left, right, part, p, color=None, cp=None):
        cd, cs, cnt, deg = part
        bias = p['fml_b'] + c_vec * p['fme_W'][0]
        A, B = _ab(right, left, p['fml_W'], bias, p['fmr_W'])
        p = dict(p)
        p['fmf_W'] = p['fmf_W'].astype(jnp.bfloat16).astype(jnp.float32)
        gbc = jnp.broadcast_to(p['fmf_ln_g'][:, None], (EMB, 16))
        bbc = jnp.broadcast_to(p['fmf_ln_b'][:, None], (EMB, 16))
        SD = _sc_edge(A, B, cd, cs, cnt, gbc, bbc)
        if color is None:
            return _post_plain(SD, deg, right, p)
        return _post_color(SD, deg, right, color, p, cp)

    cp = params['color']
    part_vc = (cd_vc, cs_vc, cnt_vc, deg_vc)
    part_cv = (cd_cv, cs_cv, cnt_cv, deg_cv)
    cons = conv(var, cons, part_vc, params['conv_v_to_c'], ccol, cp)
    var = conv(cons, var, part_cv, params['conv_c_to_v'], vcol, cp)
    cons = conv(var, cons, part_vc, params['conv_v_to_c2'])
    var = conv(cons, var, part_cv, params['conv_c_to_v2'])

    out = _head(var, params['out']['W1'], params['out']['b1'],
                params['out']['W2'])
    return jnp.concatenate([out[:H, 0], out[NH:NH + H, 0]])


# speedup vs baseline: 2.8865x; 2.7621x over previous
import functools

import jax
import jax.numpy as jnp
from jax import lax
from jax.experimental import pallas as pl
from jax.experimental.pallas import tpu as pltpu
from jax.experimental.pallas import tpu_sc as plsc

EMB = 64
NUM_MOD = 32
TILE = 512

N = 50000
H = 25000
NH = 25088
N2 = 2 * NH
PW = 25088
E2 = 32 * PW
CH = 1568
GP = CH // 16
EW = PW + 128
TGD = NH + H
CHK = 48

_HI = jax.lax.Precision.HIGHEST


def _bfdot(a, w):
    return jnp.dot(a.astype(jnp.bfloat16), w.astype(jnp.bfloat16),
                   preferred_element_type=jnp.float32)

_mesh = plsc.VectorSubcoreMesh(core_axis_name="c", subcore_axis_name="s")


def _sc_partition(dstp, srcp):
    outs = [jax.ShapeDtypeStruct((2, 32, EW), jnp.int32),
            jax.ShapeDtypeStruct((2, 32, EW), jnp.int32),
            jax.ShapeDtypeStruct((2, 32, 16), jnp.int32)]

    @functools.partial(
        pl.kernel, mesh=_mesh, out_type=outs,
        compiler_params=pltpu.CompilerParams(needs_layout_passes=False, use_tc_tiling_on_sc=False),
        scratch_types=[
            pltpu.VMEM((CH,), jnp.int32),
            pltpu.VMEM((CH,), jnp.int32),
            pltpu.VMEM((EW,), jnp.int32),
            pltpu.VMEM((EW,), jnp.int32),
            pltpu.VMEM((EW,), jnp.int32),
            pltpu.VMEM((EW,), jnp.int32),
            pltpu.VMEM((16,), jnp.int32),
        ])
    def k(dst_h, src_h, cd_h, cs_h, cnt_h,
          dbuf, sbuf, cd0v, cs0v, cd1v, cs1v, cntv):
        c = lax.axis_index("c")
        s = lax.axis_index("s")
        w = s * 2 + c
        lanes = lax.iota(jnp.int32, 16)

        def chunk(kk, offs):
            pltpu.sync_copy(dst_h.at[pl.ds(w * PW + kk * CH, CH)], dbuf)
            pltpu.sync_copy(src_h.at[pl.ds(w * PW + kk * CH, CH)], sbuf)

            def grp(g, offs):
                off0, off1 = offs
                d = dbuf[pl.ds(g * 16, 16)]
                sv = sbuf[pl.ds(g * 16, 16)]
                m1 = d >= NH
                m0 = jnp.logical_not(m1)
                m0i = m0.astype(jnp.int32)
                cum = plsc.cumsum(m0i)
                exc = cum - m0i
                cnt0 = jnp.max(cum)
                pos0 = off0 + exc
                pos1 = off1 + lanes - exc
                plsc.store_scatter(cd0v, [pos0], d, mask=m0)
                plsc.store_scatter(cs0v, [pos0], sv, mask=m0)
                plsc.store_scatter(cd1v, [pos1], d, mask=m1)
                plsc.store_scatter(cs1v, [pos1], sv, mask=m1)
                return off0 + cnt0, off1 + (16 - cnt0)

            return lax.fori_loop(0, GP, grp, offs)

        off0, off1 = lax.fori_loop(0, PW // CH, chunk,
                                   (jnp.int32(0), jnp.int32(0)))

        tg = jnp.zeros((16,), jnp.int32) + TGD
        zz = jnp.zeros((16,), jnp.int32)
        for g in range(16):
            i0 = off0 + g * 16 + lanes
            i1 = off1 + g * 16 + lanes
            plsc.store_scatter(cd0v, [i0], tg)
            plsc.store_scatter(cs0v, [i0], zz)
            plsc.store_scatter(cd1v, [i1], tg)
            plsc.store_scatter(cs1v, [i1], zz)

        pltpu.sync_copy(cd0v, cd_h.at[0, w])
        pltpu.sync_copy(cs0v, cs_h.at[0, w])
        pltpu.sync_copy(cd1v, cd_h.at[1, w])
        pltpu.sync_copy(cs1v, cs_h.at[1, w])
        cntv[...] = zz + off0
        pltpu.sync_copy(cntv, cnt_h.at[0, w])
        cntv[...] = zz + off1
        pltpu.sync_copy(cntv, cnt_h.at[1, w])

    return k(dstp, srcp)


def _sc_edge(A, B, cd, cs, cnt, gbc, bbc):
    @functools.partial(
        pl.kernel, mesh=_mesh,
        out_type=jax.ShapeDtypeStruct((N2, EMB), jnp.float32),
        compiler_params=pltpu.CompilerParams(needs_layout_passes=False, use_tc_tiling_on_sc=False),
        scratch_types=[
            [pltpu.VMEM((CHK,), jnp.int32)] * 2,
            [pltpu.VMEM((CHK,), jnp.int32)] * 2,
            [pltpu.VMEM((CHK,), jnp.int32)] * 2,
            [pltpu.VMEM((CHK, EMB), jnp.float32)] * 2,
            [pltpu.VMEM((CHK, EMB), jnp.float32)] * 2,
            [pltpu.VMEM((CHK, EMB), jnp.float32)] * 2,
            pltpu.VMEM((EMB,), jnp.float32),
            pltpu.VMEM((EMB,), jnp.float32),
            pltpu.VMEM((16,), jnp.int32),
            pltpu.VMEM_SHARED((NH, EMB), jnp.float32),
            [pltpu.SemaphoreType.DMA] * 2,
            [pltpu.SemaphoreType.DMA] * 2,
            [pltpu.SemaphoreType.DMA] * 2,
        ])
    def k(A_h, B_h, cd_h, cs_h, cnt_h, g_h, b_h, out_h,
          dsl, ssl, lidx, arows, brows, rbuf, gvm2, bvm2, cntv,
          acc, semi, semg, semsc):
        c = lax.axis_index("c")
        s = lax.axis_index("s")
        lanes = lax.iota(jnp.int32, 16)
        zf = jnp.zeros((16,), jnp.float32)

        pltpu.sync_copy(g_h, gvm2)
        pltpu.sync_copy(b_h, bvm2)

        def zr(r, _):
            for j in range(4):
                rbuf[0][r, pl.ds(j * 16, 16)] = zf
                rbuf[1][r, pl.ds(j * 16, 16)] = zf
            return 0
        lax.fori_loop(0, CHK, zr, 0)
        row0 = s * 1568
        for i in range(32):
            pltpu.sync_copy(rbuf[i % 2], acc.at[pl.ds(row0 + i * CHK, CHK)])
        pltpu.sync_copy(rbuf[0].at[pl.ds(0, 32)], acc.at[pl.ds(row0 + 1536, 32)])
        plsc.subcore_barrier()

        def idx_start(w, kk, b):
            pltpu.make_async_copy(cd_h.at[c, w, pl.ds(kk * CHK, CHK)],
                                  dsl[b], semi[b]).start()
            pltpu.make_async_copy(cs_h.at[c, w, pl.ds(kk * CHK, CHK)],
                                  ssl[b], semi[b]).start()

        def idx_wait(w, kk, b):
            pltpu.make_async_copy(cd_h.at[c, w, pl.ds(kk * CHK, CHK)],
                                  dsl[b], semi[b]).wait()
            pltpu.make_async_copy(cs_h.at[c, w, pl.ds(kk * CHK, CHK)],
                                  ssl[b], semi[b]).wait()

        def gather_start(b):
            pltpu.make_async_copy(A_h.at[dsl[b]], arows[b], semg[b]).start()
            pltpu.make_async_copy(B_h.at[ssl[b]], brows[b], semg[b]).start()

        def gather_wait(b):
            pltpu.make_async_copy(A_h.at[dsl[b]], arows[b], semg[b]).wait()
            pltpu.make_async_copy(B_h.at[ssl[b]], brows[b], semg[b]).wait()

        def scat_start(b):
            pltpu.make_async_copy(rbuf[b], acc.at[lidx[b]], semsc[b]).start(add=True)

        def scat_wait(b):
            pltpu.make_async_copy(rbuf[b], acc.at[lidx[b]], semsc[b]).wait()

        def compute_chunk(b):
            def edges4(q, _):
                for u in range(4):
                    e = q * 4 + u
                    a0 = arows[b][e, pl.ds(0, 16)]
                    a1 = arows[b][e, pl.ds(16, 16)]
                    a2 = arows[b][e, pl.ds(32, 16)]
                    a3 = arows[b][e, pl.ds(48, 16)]
                    b0 = brows[b][e, pl.ds(0, 16)]
                    b1 = brows[b][e, pl.ds(16, 16)]
                    b2 = brows[b][e, pl.ds(32, 16)]
                    b3 = brows[b][e, pl.ds(48, 16)]
                    p0 = a0 + b0
                    p1 = a1 + b1
                    p2 = a2 + b2
                    p3 = a3 + b3
                    sv = (p0 + p1) + (p2 + p3)
                    qv = (p0 * p0 + p1 * p1) + (p2 * p2 + p3 * p3)
                    sm = jnp.sum(sv)
                    sq = jnp.sum(qv)
                    mean = sm * (1.0 / EMB)
                    var = zf + (sq * (1.0 / EMB) - mean * mean + 1e-5)
                    ih = plsc.bitcast(var, jnp.int32)
                    ih = 0x5F3759DF - (ih >> 1)
                    y = plsc.bitcast(ih, jnp.float32)
                    for _ in range(3):
                        y = y * (1.5 - 0.5 * var * y * y)
                    for j, pj in enumerate((p0, p1, p2, p3)):
                        o = (pj - mean) * y * gvm2[pl.ds(j * 16, 16)]                             + bvm2[pl.ds(j * 16, 16)]
                        o = jnp.maximum(o, 0.0)
                        oi = plsc.bitcast(o, jnp.int32)
                        oi = oi + 0x7FFF + ((oi >> 16) & 1)
                        oi = jnp.bitwise_and(oi, -65536)
                        rbuf[b][e, pl.ds(j * 16, 16)] = plsc.bitcast(oi, jnp.float32)
                return 0
            lax.fori_loop(0, CHK // 4, edges4, 0)

        def do_worker(w):
            pltpu.sync_copy(cnt_h.at[c, w], cntv)
            cntw = jnp.max(cntv[...])
            nk = ((cntw + 2 * CHK - 1) // (2 * CHK)) * 2

            @pl.when(nk > 0)
            def _prime():
                pltpu.sync_copy(cd_h.at[c, w, pl.ds(0, CHK)], dsl[0])
                pltpu.sync_copy(cs_h.at[c, w, pl.ds(0, CHK)], ssl[0])
                gather_start(0)
                idx_start(w, 1, 1)

            def pair(jj, _):
                for b in range(2):
                    kk = 2 * jj + b
                    gather_wait(b)

                    @pl.when(kk >= 2)
                    def _():
                        scat_wait(b)
                    compute_lidx_and_rows(w, kk, b)
                return 0

            def compute_lidx_and_rows(w, kk, b):
                def li(g, _):
                    d = dsl[b][pl.ds(g * 16, 16)]
                    lr = d - c * NH
                    ok = jnp.logical_and(lr >= 0, lr < H)
                    lidx[b][pl.ds(g * 16, 16)] = jnp.where(ok, lr, H)
                    return 0
                lax.fori_loop(0, CHK // 16, li, 0)

                @pl.when(kk + 2 < nk)
                def _():
                    idx_start(w, kk + 2, b)

                @pl.when(kk + 1 < nk)
                def _():
                    idx_wait(w, kk + 1, 1 - b)
                    gather_start(1 - b)

                compute_chunk(b)
                scat_start(b)

            lax.fori_loop(0, nk // 2, pair, 0)

            @pl.when(nk >= 2)
            def _drain():
                scat_wait(0)
                scat_wait(1)

        do_worker(2 * s)
        do_worker(2 * s + 1)
        plsc.subcore_barrier()
        pltpu.sync_copy(acc.at[pl.ds(row0, 1568)],
                        out_h.at[pl.ds(c * NH + row0, 1568)])

    return k(A, B, cd, cs, cnt, gbc, bbc)


def _sc_deg(cd, cnt):
    @functools.partial(
        pl.kernel, mesh=_mesh,
        out_type=jax.ShapeDtypeStruct((N2, 16), jnp.float32),
        compiler_params=pltpu.CompilerParams(needs_layout_passes=False, use_tc_tiling_on_sc=False),
        scratch_types=[
            pltpu.VMEM((128,), jnp.int32),
            pltpu.VMEM((128,), jnp.int32),
            pltpu.VMEM((128, 16), jnp.float32),
            pltpu.VMEM((16,), jnp.int32),
            pltpu.VMEM_SHARED((NH, 16), jnp.float32),
        ])
    def k(cd_h, cnt_h, out_h, dvm, lidxv, obuf, cntv, dacc):
        c = lax.axis_index("c")
        s = lax.axis_index("s")
        zf = jnp.zeros((16,), jnp.float32)

        def zr(r, _):
            obuf[r] = zf
            return 0
        lax.fori_loop(0, 128, zr, 0)
        row0 = s * 1568
        for i in range(12):
            pltpu.sync_copy(obuf, dacc.at[pl.ds(row0 + i * 128, 128)])
        pltpu.sync_copy(obuf.at[pl.ds(0, 32)], dacc.at[pl.ds(row0 + 1536, 32)])

        def o1(r, _):
            obuf[r] = zf + 1.0
            return 0
        lax.fori_loop(0, 128, o1, 0)
        plsc.subcore_barrier()

        def do_worker(w):
            pltpu.sync_copy(cnt_h.at[c, w], cntv)
            nk = (jnp.max(cntv[...]) + 127) // 128

            def chunk(kk, _):
                pltpu.sync_copy(cd_h.at[c, w, pl.ds(kk * 128, 128)], dvm)

                def li(g, _):
                    d = dvm[pl.ds(g * 16, 16)]
                    lr = d - c * NH
                    ok = jnp.logical_and(lr >= 0, lr < H)
                    lidxv[pl.ds(g * 16, 16)] = jnp.where(ok, lr, H)
                    return 0
                lax.fori_loop(0, 8, li, 0)
                pltpu.sync_copy(obuf, dacc.at[lidxv], add=True)
                return 0
            lax.fori_loop(0, nk, chunk, 0)

        do_worker(2 * s)
        do_worker(2 * s + 1)
        plsc.subcore_barrier()
        pltpu.sync_copy(dacc.at[pl.ds(row0, 1568)],
                        out_h.at[pl.ds(c * NH + row0, 1568)])

    return k(cd, cnt)


def _ln_rows(x, g, b, eps=1e-5):
    m = jnp.mean(x, axis=-1, keepdims=True)
    v = jnp.mean((x - m) ** 2, axis=-1, keepdims=True)
    return (x - m) / jnp.sqrt(v + eps) * g + b


def _emb2_kernel(x_ref, w1_ref, b1_ref, w2_ref, b2_ref, g_ref, bb_ref, o_ref):
    x = x_ref[...]
    h = _ln_rows(x, g_ref[...], bb_ref[...])
    h = jax.nn.relu(_bfdot(h, w1_ref[...]) + b1_ref[...])
    o_ref[...] = jax.nn.relu(_bfdot(h, w2_ref[...]) + b2_ref[...])


def _emb2(x, p):
    F = x.shape[1]
    grid = N2 // TILE
    return pl.pallas_call(
        _emb2_kernel,
        grid=(grid,),
        in_specs=[
            pl.BlockSpec((TILE, F), lambda i: (i, 0)),
            pl.BlockSpec((F, EMB), lambda i: (0, 0)),
            pl.BlockSpec((EMB,), lambda i: (0,)),
            pl.BlockSpec((EMB, EMB), lambda i: (0, 0)),
            pl.BlockSpec((EMB,), lambda i: (0,)),
            pl.BlockSpec((F,), lambda i: (0,)),
            pl.BlockSpec((F,), lambda i: (0,)),
        ],
        out_specs=pl.BlockSpec((TILE, EMB), lambda i: (i, 0)),
        out_shape=jax.ShapeDtypeStruct((N2, EMB), jnp.float32),
    )(x, p['W1'], p['b1'], p['W2'], p['b2'], p['ln_g'], p['ln_b'])


def _ab_kernel(r_ref, l_ref, wl_ref, bl_ref, wr_ref, a_ref, b_ref):
    a_ref[...] = _bfdot(r_ref[...], wl_ref[...]) + bl_ref[...]
    b_ref[...] = _bfdot(l_ref[...], wr_ref[...])


def _ab(right, left, wl, bl, wr):
    grid = N2 // TILE
    return pl.pallas_call(
        _ab_kernel,
        grid=(grid,),
        in_specs=[
            pl.BlockSpec((TILE, EMB), lambda i: (i, 0)),
            pl.BlockSpec((TILE, EMB), lambda i: (i, 0)),
            pl.BlockSpec((EMB, EMB), lambda i: (0, 0)),
            pl.BlockSpec((EMB,), lambda i: (0,)),
            pl.BlockSpec((EMB, EMB), lambda i: (0, 0)),
        ],
        out_specs=[
            pl.BlockSpec((TILE, EMB), lambda i: (i, 0)),
            pl.BlockSpec((TILE, EMB), lambda i: (i, 0)),
        ],
        out_shape=[
            jax.ShapeDtypeStruct((N2, EMB), jnp.float32),
            jax.ShapeDtypeStruct((N2, EMB), jnp.float32),
        ],
    )(right, left, wl, bl, wr)


def _agg_cat(sd, deg, rt, fw, fb, pg, pb):
    agg = (jnp.dot(sd, fw, preferred_element_type=jnp.float32,
                   precision=_HI) + deg[:, :1] * fb)
    aggl = _ln_rows(agg, pg, pb)
    return jnp.concatenate([aggl, rt], axis=-1)


def _post_plain_kernel(sd_ref, deg_ref, rt_ref, fw_ref, fb_ref, pg_ref, pb_ref,
                       w1_ref, b1_ref, w2_ref, b2_ref, o_ref):
    cat = _agg_cat(sd_ref[...], deg_ref[...], rt_ref[...], fw_ref[...],
                   fb_ref[...], pg_ref[...], pb_ref[...])
    h = jax.nn.relu(_bfdot(cat, w1_ref[...]) + b1_ref[...])
    o_ref[...] = _bfdot(h, w2_ref[...]) + b2_ref[...]


def _post_plain(SD, deg, right, p):
    grid = N2 // TILE
    return pl.pallas_call(
        _post_plain_kernel,
        grid=(grid,),
        in_specs=[
            pl.BlockSpec((TILE, EMB), lambda i: (i, 0)),
            pl.BlockSpec((TILE, 16), lambda i: (i, 0)),
            pl.BlockSpec((TILE, EMB), lambda i: (i, 0)),
            pl.BlockSpec((EMB, EMB), lambda i: (0, 0)),
            pl.BlockSpec((EMB,), lambda i: (0,)),
            pl.BlockSpec((EMB,), lambda i: (0,)),
            pl.BlockSpec((EMB,), lambda i: (0,)),
            pl.BlockSpec((2 * EMB, EMB), lambda i: (0, 0)),
            pl.BlockSpec((EMB,), lambda i: (0,)),
            pl.BlockSpec((EMB, EMB), lambda i: (0, 0)),
            pl.BlockSpec((EMB,), lambda i: (0,)),
        ],
        out_specs=pl.BlockSpec((TILE, EMB), lambda i: (i, 0)),
        out_shape=jax.ShapeDtypeStruct((N2, EMB), jnp.float32),
    )(SD, deg, right, p['fmf_W'], p['fmf_b'], p['post_ln_g'], p['post_ln_b'],
      p['out_W1'], p['out_b1'], p['out_W2'], p['out_b2'])


def _post_color_kernel(sd_ref, deg_ref, rt_ref, col_ref, fw_ref, fb_ref,
                       pg_ref, pb_ref, w1_ref, b1_ref, w2_ref, b2_ref, o_ref):
    cat = _agg_cat(sd_ref[...], deg_ref[...], rt_ref[...], fw_ref[...],
                   fb_ref[...], pg_ref[...], pb_ref[...])
    col = col_ref[...]
    acc = jnp.zeros((cat.shape[0], EMB), jnp.float32)
    catb = cat.astype(jnp.bfloat16)
    for c in range(NUM_MOD):
        h = jax.nn.relu(jnp.dot(catb, w1_ref[c].astype(jnp.bfloat16),
                                preferred_element_type=jnp.float32) + b1_ref[c])
        z = _bfdot(h, w2_ref[c]) + b2_ref[c]
        acc = jnp.where(col == c, z, acc)
    o_ref[...] = acc


def _post_color(SD, deg, right, color, p, cp):
    grid = N2 // TILE
    return pl.pallas_call(
        _post_color_kernel,
        grid=(grid,),
        in_specs=[
            pl.BlockSpec((TILE, EMB), lambda i: (i, 0)),
            pl.BlockSpec((TILE, 16), lambda i: (i, 0)),
            pl.BlockSpec((TILE, EMB), lambda i: (i, 0)),
            pl.BlockSpec((TILE, 1), lambda i: (i, 0)),
            pl.BlockSpec((EMB, EMB), lambda i: (0, 0)),
            pl.BlockSpec((EMB,), lambda i: (0,)),
            pl.BlockSpec((EMB,), lambda i: (0,)),
            pl.BlockSpec((EMB,), lambda i: (0,)),
            pl.BlockSpec((NUM_MOD, 2 * EMB, EMB), lambda i: (0, 0, 0)),
            pl.BlockSpec((NUM_MOD, EMB), lambda i: (0, 0)),
            pl.BlockSpec((NUM_MOD, EMB, EMB), lambda i: (0, 0, 0)),
            pl.BlockSpec((NUM_MOD, EMB), lambda i: (0, 0)),
        ],
        out_specs=pl.BlockSpec((TILE, EMB), lambda i: (i, 0)),
        out_shape=jax.ShapeDtypeStruct((N2, EMB), jnp.float32),
    )(SD, deg, right, color, p['fmf_W'], p['fmf_b'], p['post_ln_g'],
      p['post_ln_b'], cp['W1'], cp['b1'], cp['W2'], cp['b2'])


def _head_kernel(x_ref, w1_ref, b1_ref, w2_ref, o_ref):
    h = jax.nn.relu(_bfdot(x_ref[...], w1_ref[...]) + b1_ref[...])
    o_ref[...] = _bfdot(h, w2_ref[...])


def _head(x, w1, b1, w2):
    grid = N2 // TILE
    w2p = jnp.pad(w2, ((0, 0), (0, 127)))
    return pl.pallas_call(
        _head_kernel,
        grid=(grid,),
        in_specs=[
            pl.BlockSpec((TILE, EMB), lambda i: (i, 0)),
            pl.BlockSpec((EMB, EMB), lambda i: (0, 0)),
            pl.BlockSpec((EMB,), lambda i: (0,)),
            pl.BlockSpec((EMB, 128), lambda i: (0, 0)),
        ],
        out_specs=pl.BlockSpec((TILE, 128), lambda i: (i, 0)),
        out_shape=jax.ShapeDtypeStruct((N2, 128), jnp.float32),
    )(x, w1, b1, w2p)


def _to_layout(x):
    z = jnp.zeros((NH - H,) + x.shape[1:], x.dtype)
    return jnp.concatenate([x[:H], z, x[H:], z], axis=0)


def kernel(constraint_features, edge_indices, edge_features, variable_features,
           variableColor, consColor, params):
    cons = _emb2(_to_layout(constraint_features), params['cons_emb'])
    var = _emb2(_to_layout(variable_features), params['var_emb'])

    c_vec = params['edge_ln']['b'][0]

    ccol = _to_layout(consColor)[:, None]
    vcol = _to_layout(variableColor)[:, None]

    ec = edge_indices[0]
    ev = edge_indices[1]
    ecp = ec + (ec >= H).astype(jnp.int32) * (NH - H)
    evp = ev + (ev >= H).astype(jnp.int32) * (NH - H)
    npad_e = E2 - ec.shape[0]
    padd = jnp.full((npad_e,), TGD, jnp.int32)
    padz = jnp.zeros((npad_e,), jnp.int32)

    cd_vc, cs_vc, cnt_vc = _sc_partition(jnp.concatenate([ecp, padd]),
                                         jnp.concatenate([evp, padz]))
    cd_cv, cs_cv, cnt_cv = _sc_partition(jnp.concatenate([evp, padd]),
                                         jnp.concatenate([ecp, padz]))
    deg_vc = _sc_deg(cd_vc, cnt_vc)
    deg_cv = _sc_deg(cd_cv, cnt_cv)

    def conv(left, right, part, p, color=None, cp=None):
        cd, cs, cnt, deg = part
        bias = p['fml_b'] + c_vec * p['fme_W'][0]
        A, B = _ab(right, left, p['fml_W'], bias, p['fmr_W'])
        p = dict(p)
        p['fmf_W'] = p['fmf_W'].astype(jnp.bfloat16).astype(jnp.float32)
        SD = _sc_edge(A, B, cd, cs, cnt, p['fmf_ln_g'], p['fmf_ln_b'])
        if color is None:
            return _post_plain(SD, deg, right, p)
        return _post_color(SD, deg, right, color, p, cp)

    cp = params['color']
    part_vc = (cd_vc, cs_vc, cnt_vc, deg_vc)
    part_cv = (cd_cv, cs_cv, cnt_cv, deg_cv)
    cons = conv(var, cons, part_vc, params['conv_v_to_c'], ccol, cp)
    var = conv(cons, var, part_cv, params['conv_c_to_v'], vcol, cp)
    cons = conv(var, cons, part_vc, params['conv_v_to_c2'])
    var = conv(cons, var, part_cv, params['conv_c_to_v2'])

    out = _head(var, params['out']['W1'], params['out']['b1'],
                params['out']['W2'])
    return jnp.concatenate([out[:H, 0], out[NH:NH + H, 0]])


# speedup vs baseline: 3.1925x; 1.1060x over previous
import functools

import jax
import jax.numpy as jnp
from jax import lax
from jax.experimental import pallas as pl
from jax.experimental.pallas import tpu as pltpu
from jax.experimental.pallas import tpu_sc as plsc

EMB = 64
NUM_MOD = 32
TILE = 512

N = 50000
H = 25000
NH = 25088
N2 = 2 * NH
PW = 25088
E2 = 32 * PW
CH = 1568
GP = CH // 16
EW = PW + 128
TGD = NH + H
CHK = 64

_HI = jax.lax.Precision.HIGHEST


def _bfdot(a, w):
    return jnp.dot(a.astype(jnp.bfloat16), w.astype(jnp.bfloat16),
                   preferred_element_type=jnp.float32)

_mesh = plsc.VectorSubcoreMesh(core_axis_name="c", subcore_axis_name="s")


def _sc_partition(dstp, srcp):
    outs = [jax.ShapeDtypeStruct((2, 32, EW), jnp.int32),
            jax.ShapeDtypeStruct((2, 32, EW), jnp.int32),
            jax.ShapeDtypeStruct((2, 32, 16), jnp.int32)]

    @functools.partial(
        pl.kernel, mesh=_mesh, out_type=outs,
        compiler_params=pltpu.CompilerParams(needs_layout_passes=False, use_tc_tiling_on_sc=False),
        scratch_types=[
            pltpu.VMEM((CH,), jnp.int32),
            pltpu.VMEM((CH,), jnp.int32),
            pltpu.VMEM((EW,), jnp.int32),
            pltpu.VMEM((EW,), jnp.int32),
            pltpu.VMEM((EW,), jnp.int32),
            pltpu.VMEM((EW,), jnp.int32),
            pltpu.VMEM((16,), jnp.int32),
        ])
    def k(dst_h, src_h, cd_h, cs_h, cnt_h,
          dbuf, sbuf, cd0v, cs0v, cd1v, cs1v, cntv):
        c = lax.axis_index("c")
        s = lax.axis_index("s")
        w = s * 2 + c
        lanes = lax.iota(jnp.int32, 16)

        def chunk(kk, offs):
            pltpu.sync_copy(dst_h.at[pl.ds(w * PW + kk * CH, CH)], dbuf)
            pltpu.sync_copy(src_h.at[pl.ds(w * PW + kk * CH, CH)], sbuf)

            def grp(g, offs):
                off0, off1 = offs
                d = dbuf[pl.ds(g * 16, 16)]
                sv = sbuf[pl.ds(g * 16, 16)]
                m1 = d >= NH
                m0 = jnp.logical_not(m1)
                m0i = m0.astype(jnp.int32)
                cum = plsc.cumsum(m0i)
                exc = cum - m0i
                cnt0 = jnp.max(cum)
                pos0 = off0 + exc
                pos1 = off1 + lanes - exc
                plsc.store_scatter(cd0v, [pos0], d, mask=m0)
                plsc.store_scatter(cs0v, [pos0], sv, mask=m0)
                plsc.store_scatter(cd1v, [pos1], d, mask=m1)
                plsc.store_scatter(cs1v, [pos1], sv, mask=m1)
                return off0 + cnt0, off1 + (16 - cnt0)

            return lax.fori_loop(0, GP, grp, offs)

        off0, off1 = lax.fori_loop(0, PW // CH, chunk,
                                   (jnp.int32(0), jnp.int32(0)))

        tg = jnp.zeros((16,), jnp.int32) + TGD
        zz = jnp.zeros((16,), jnp.int32)
        for g in range(16):
            i0 = off0 + g * 16 + lanes
            i1 = off1 + g * 16 + lanes
            plsc.store_scatter(cd0v, [i0], tg)
            plsc.store_scatter(cs0v, [i0], zz)
            plsc.store_scatter(cd1v, [i1], tg)
            plsc.store_scatter(cs1v, [i1], zz)

        pltpu.sync_copy(cd0v, cd_h.at[0, w])
        pltpu.sync_copy(cs0v, cs_h.at[0, w])
        pltpu.sync_copy(cd1v, cd_h.at[1, w])
        pltpu.sync_copy(cs1v, cs_h.at[1, w])
        cntv[...] = zz + off0
        pltpu.sync_copy(cntv, cnt_h.at[0, w])
        cntv[...] = zz + off1
        pltpu.sync_copy(cntv, cnt_h.at[1, w])

    return k(dstp, srcp)


def _sc_edge(A, B, cd, cs, cnt, gbc, bbc):
    @functools.partial(
        pl.kernel, mesh=_mesh,
        out_type=jax.ShapeDtypeStruct((N2, EMB), jnp.float32),
        compiler_params=pltpu.CompilerParams(needs_layout_passes=False, use_tc_tiling_on_sc=False),
        scratch_types=[
            [pltpu.VMEM((CHK,), jnp.int32)] * 2,
            [pltpu.VMEM((CHK,), jnp.int32)] * 2,
            [pltpu.VMEM((CHK,), jnp.int32)] * 2,
            [pltpu.VMEM((CHK, EMB), jnp.float32)] * 2,
            [pltpu.VMEM((CHK, EMB), jnp.float32)] * 2,
            [pltpu.VMEM((CHK, EMB), jnp.float32)] * 2,
            pltpu.VMEM((EMB,), jnp.float32),
            pltpu.VMEM((EMB,), jnp.float32),
            pltpu.VMEM((16,), jnp.int32),
            pltpu.VMEM_SHARED((NH, EMB), jnp.float32),
            [pltpu.SemaphoreType.DMA] * 2,
            [pltpu.SemaphoreType.DMA] * 2,
            [pltpu.SemaphoreType.DMA] * 2,
        ])
    def k(A_h, B_h, cd_h, cs_h, cnt_h, g_h, b_h, out_h,
          dsl, ssl, lidx, arows, brows, rbuf, gvm2, bvm2, cntv,
          acc, semi, semg, semsc):
        c = lax.axis_index("c")
        s = lax.axis_index("s")
        lanes = lax.iota(jnp.int32, 16)
        zf = jnp.zeros((16,), jnp.float32)

        pltpu.sync_copy(g_h, gvm2)
        pltpu.sync_copy(b_h, bvm2)

        def zr(r, _):
            for j in range(4):
                rbuf[0][r, pl.ds(j * 16, 16)] = zf
                rbuf[1][r, pl.ds(j * 16, 16)] = zf
            return 0
        lax.fori_loop(0, CHK, zr, 0)
        row0 = s * 1568
        for i in range(24):
            pltpu.sync_copy(rbuf[i % 2], acc.at[pl.ds(row0 + i * CHK, CHK)])
        pltpu.sync_copy(rbuf[0].at[pl.ds(0, 32)], acc.at[pl.ds(row0 + 1536, 32)])
        plsc.subcore_barrier()

        def idx_start(w, kk, b):
            pltpu.make_async_copy(cd_h.at[c, w, pl.ds(kk * CHK, CHK)],
                                  dsl[b], semi[b]).start()
            pltpu.make_async_copy(cs_h.at[c, w, pl.ds(kk * CHK, CHK)],
                                  ssl[b], semi[b]).start()

        def idx_wait(w, kk, b):
            pltpu.make_async_copy(cd_h.at[c, w, pl.ds(kk * CHK, CHK)],
                                  dsl[b], semi[b]).wait()
            pltpu.make_async_copy(cs_h.at[c, w, pl.ds(kk * CHK, CHK)],
                                  ssl[b], semi[b]).wait()

        def gather_start(b):
            pltpu.make_async_copy(A_h.at[dsl[b]], arows[b], semg[b]).start()
            pltpu.make_async_copy(B_h.at[ssl[b]], brows[b], semg[b]).start()

        def gather_wait(b):
            pltpu.make_async_copy(A_h.at[dsl[b]], arows[b], semg[b]).wait()
            pltpu.make_async_copy(B_h.at[ssl[b]], brows[b], semg[b]).wait()

        def scat_start(b):
            pltpu.make_async_copy(rbuf[b], acc.at[lidx[b]], semsc[b]).start(add=True)

        def scat_wait(b):
            pltpu.make_async_copy(rbuf[b], acc.at[lidx[b]], semsc[b]).wait()

        def compute_chunk(b):
            def edges4(q, _):
                for u in range(8):
                    e = q * 8 + u
                    a0 = arows[b][e, pl.ds(0, 16)]
                    a1 = arows[b][e, pl.ds(16, 16)]
                    a2 = arows[b][e, pl.ds(32, 16)]
                    a3 = arows[b][e, pl.ds(48, 16)]
                    b0 = brows[b][e, pl.ds(0, 16)]
                    b1 = brows[b][e, pl.ds(16, 16)]
                    b2 = brows[b][e, pl.ds(32, 16)]
                    b3 = brows[b][e, pl.ds(48, 16)]
                    p0 = a0 + b0
                    p1 = a1 + b1
                    p2 = a2 + b2
                    p3 = a3 + b3
                    sv = (p0 + p1) + (p2 + p3)
                    qv = (p0 * p0 + p1 * p1) + (p2 * p2 + p3 * p3)
                    sm = jnp.sum(sv)
                    sq = jnp.sum(qv)
                    mean = sm * (1.0 / EMB)
                    var = zf + (sq * (1.0 / EMB) - mean * mean + 1e-5)
                    ih = plsc.bitcast(var, jnp.int32)
                    ih = 0x5F3759DF - (ih >> 1)
                    y = plsc.bitcast(ih, jnp.float32)
                    for _ in range(3):
                        y = y * (1.5 - 0.5 * var * y * y)
                    for j, pj in enumerate((p0, p1, p2, p3)):
                        o = (pj - mean) * y * gvm2[pl.ds(j * 16, 16)]                             + bvm2[pl.ds(j * 16, 16)]
                        o = jnp.maximum(o, 0.0)
                        oi = plsc.bitcast(o, jnp.int32)
                        oi = oi + 0x7FFF + ((oi >> 16) & 1)
                        oi = jnp.bitwise_and(oi, -65536)
                        rbuf[b][e, pl.ds(j * 16, 16)] = plsc.bitcast(oi, jnp.float32)
                return 0
            lax.fori_loop(0, CHK // 8, edges4, 0)

        def do_worker(w):
            pltpu.sync_copy(cnt_h.at[c, w], cntv)
            cntw = jnp.max(cntv[...])
            nk = ((cntw + 2 * CHK - 1) // (2 * CHK)) * 2

            @pl.when(nk > 0)
            def _prime():
                pltpu.sync_copy(cd_h.at[c, w, pl.ds(0, CHK)], dsl[0])
                pltpu.sync_copy(cs_h.at[c, w, pl.ds(0, CHK)], ssl[0])
                gather_start(0)
                idx_start(w, 1, 1)

            def pair(jj, _):
                for b in range(2):
                    kk = 2 * jj + b
                    gather_wait(b)

                    @pl.when(kk >= 2)
                    def _():
                        scat_wait(b)
                    compute_lidx_and_rows(w, kk, b)
                return 0

            def compute_lidx_and_rows(w, kk, b):
                def li(g, _):
                    d = dsl[b][pl.ds(g * 16, 16)]
                    lr = d - c * NH
                    ok = jnp.logical_and(lr >= 0, lr < H)
                    lidx[b][pl.ds(g * 16, 16)] = jnp.where(ok, lr, H)
                    return 0
                lax.fori_loop(0, CHK // 16, li, 0)

                @pl.when(kk + 2 < nk)
                def _():
                    idx_start(w, kk + 2, b)

                @pl.when(kk + 1 < nk)
                def _():
                    idx_wait(w, kk + 1, 1 - b)
                    gather_start(1 - b)

                compute_chunk(b)
                scat_start(b)

            lax.fori_loop(0, nk // 2, pair, 0)

            @pl.when(nk >= 2)
            def _drain():
                scat_wait(0)
                scat_wait(1)

        do_worker(2 * s)
        do_worker(2 * s + 1)
        plsc.subcore_barrier()
        pltpu.sync_copy(acc.at[pl.ds(row0, 1568)],
                        out_h.at[pl.ds(c * NH + row0, 1568)])

    return k(A, B, cd, cs, cnt, gbc, bbc)


def _sc_deg(cd, cnt):
    @functools.partial(
        pl.kernel, mesh=_mesh,
        out_type=jax.ShapeDtypeStruct((N2, 16), jnp.float32),
        compiler_params=pltpu.CompilerParams(needs_layout_passes=False, use_tc_tiling_on_sc=False),
        scratch_types=[
            pltpu.VMEM((128,), jnp.int32),
            pltpu.VMEM((128,), jnp.int32),
            pltpu.VMEM((128, 16), jnp.float32),
            pltpu.VMEM((16,), jnp.int32),
            pltpu.VMEM_SHARED((NH, 16), jnp.float32),
        ])
    def k(cd_h, cnt_h, out_h, dvm, lidxv, obuf, cntv, dacc):
        c = lax.axis_index("c")
        s = lax.axis_index("s")
        zf = jnp.zeros((16,), jnp.float32)

        def zr(r, _):
            obuf[r] = zf
            return 0
        lax.fori_loop(0, 128, zr, 0)
        row0 = s * 1568
        for i in range(12):
            pltpu.sync_copy(obuf, dacc.at[pl.ds(row0 + i * 128, 128)])
        pltpu.sync_copy(obuf.at[pl.ds(0, 32)], dacc.at[pl.ds(row0 + 1536, 32)])

        def o1(r, _):
            obuf[r] = zf + 1.0
            return 0
        lax.fori_loop(0, 128, o1, 0)
        plsc.subcore_barrier()

        def do_worker(w):
            pltpu.sync_copy(cnt_h.at[c, w], cntv)
            nk = (jnp.max(cntv[...]) + 127) // 128

            def chunk(kk, _):
                pltpu.sync_copy(cd_h.at[c, w, pl.ds(kk * 128, 128)], dvm)

                def li(g, _):
                    d = dvm[pl.ds(g * 16, 16)]
                    lr = d - c * NH
                    ok = jnp.logical_and(lr >= 0, lr < H)
                    lidxv[pl.ds(g * 16, 16)] = jnp.where(ok, lr, H)
                    return 0
                lax.fori_loop(0, 8, li, 0)
                pltpu.sync_copy(obuf, dacc.at[lidxv], add=True)
                return 0
            lax.fori_loop(0, nk, chunk, 0)

        do_worker(2 * s)
        do_worker(2 * s + 1)
        plsc.subcore_barrier()
        pltpu.sync_copy(dacc.at[pl.ds(row0, 1568)],
                        out_h.at[pl.ds(c * NH + row0, 1568)])

    return k(cd, cnt)


def _ln_rows(x, g, b, eps=1e-5):
    m = jnp.mean(x, axis=-1, keepdims=True)
    v = jnp.mean((x - m) ** 2, axis=-1, keepdims=True)
    return (x - m) / jnp.sqrt(v + eps) * g + b


def _emb2_kernel(x_ref, w1_ref, b1_ref, w2_ref, b2_ref, g_ref, bb_ref, o_ref):
    x = x_ref[...]
    h = _ln_rows(x, g_ref[...], bb_ref[...])
    h = jax.nn.relu(_bfdot(h, w1_ref[...]) + b1_ref[...])
    o_ref[...] = jax.nn.relu(_bfdot(h, w2_ref[...]) + b2_ref[...])


def _emb2(x, p):
    F = x.shape[1]
    grid = N2 // TILE
    return pl.pallas_call(
        _emb2_kernel,
        grid=(grid,),
        in_specs=[
            pl.BlockSpec((TILE, F), lambda i: (i, 0)),
            pl.BlockSpec((F, EMB), lambda i: (0, 0)),
            pl.BlockSpec((EMB,), lambda i: (0,)),
            pl.BlockSpec((EMB, EMB), lambda i: (0, 0)),
            pl.BlockSpec((EMB,), lambda i: (0,)),
            pl.BlockSpec((F,), lambda i: (0,)),
            pl.BlockSpec((F,), lambda i: (0,)),
        ],
        out_specs=pl.BlockSpec((TILE, EMB), lambda i: (i, 0)),
        out_shape=jax.ShapeDtypeStruct((N2, EMB), jnp.float32),
    )(x, p['W1'], p['b1'], p['W2'], p['b2'], p['ln_g'], p['ln_b'])


def _ab_kernel(r_ref, l_ref, wl_ref, bl_ref, wr_ref, a_ref, b_ref):
    a_ref[...] = _bfdot(r_ref[...], wl_ref[...]) + bl_ref[...]
    b_ref[...] = _bfdot(l_ref[...], wr_ref[...])


def _ab(right, left, wl, bl, wr):
    grid = N2 // TILE
    return pl.pallas_call(
        _ab_kernel,
        grid=(grid,),
        in_specs=[
            pl.BlockSpec((TILE, EMB), lambda i: (i, 0)),
            pl.BlockSpec((TILE, EMB), lambda i: (i, 0)),
            pl.BlockSpec((EMB, EMB), lambda i: (0, 0)),
            pl.BlockSpec((EMB,), lambda i: (0,)),
            pl.BlockSpec((EMB, EMB), lambda i: (0, 0)),
        ],
        out_specs=[
            pl.BlockSpec((TILE, EMB), lambda i: (i, 0)),
            pl.BlockSpec((TILE, EMB), lambda i: (i, 0)),
        ],
        out_shape=[
            jax.ShapeDtypeStruct((N2, EMB), jnp.float32),
            jax.ShapeDtypeStruct((N2, EMB), jnp.float32),
        ],
    )(right, left, wl, bl, wr)


def _agg_cat(sd, deg, rt, fw, fb, pg, pb):
    agg = (jnp.dot(sd, fw, preferred_element_type=jnp.float32,
                   precision=_HI) + deg[:, :1] * fb)
    aggl = _ln_rows(agg, pg, pb)
    return jnp.concatenate([aggl, rt], axis=-1)


def _post_plain_kernel(sd_ref, deg_ref, rt_ref, fw_ref, fb_ref, pg_ref, pb_ref,
                       w1_ref, b1_ref, w2_ref, b2_ref, o_ref):
    cat = _agg_cat(sd_ref[...], deg_ref[...], rt_ref[...], fw_ref[...],
                   fb_ref[...], pg_ref[...], pb_ref[...])
    h = jax.nn.relu(_bfdot(cat, w1_ref[...]) + b1_ref[...])
    o_ref[...] = _bfdot(h, w2_ref[...]) + b2_ref[...]


def _post_plain(SD, deg, right, p):
    grid = N2 // TILE
    return pl.pallas_call(
        _post_plain_kernel,
        grid=(grid,),
        in_specs=[
            pl.BlockSpec((TILE, EMB), lambda i: (i, 0)),
            pl.BlockSpec((TILE, 16), lambda i: (i, 0)),
            pl.BlockSpec((TILE, EMB), lambda i: (i, 0)),
            pl.BlockSpec((EMB, EMB), lambda i: (0, 0)),
            pl.BlockSpec((EMB,), lambda i: (0,)),
            pl.BlockSpec((EMB,), lambda i: (0,)),
            pl.BlockSpec((EMB,), lambda i: (0,)),
            pl.BlockSpec((2 * EMB, EMB), lambda i: (0, 0)),
            pl.BlockSpec((EMB,), lambda i: (0,)),
            pl.BlockSpec((EMB, EMB), lambda i: (0, 0)),
            pl.BlockSpec((EMB,), lambda i: (0,)),
        ],
        out_specs=pl.BlockSpec((TILE, EMB), lambda i: (i, 0)),
        out_shape=jax.ShapeDtypeStruct((N2, EMB), jnp.float32),
    )(SD, deg, right, p['fmf_W'], p['fmf_b'], p['post_ln_g'], p['post_ln_b'],
      p['out_W1'], p['out_b1'], p['out_W2'], p['out_b2'])


def _post_color_kernel(sd_ref, deg_ref, rt_ref, col_ref, fw_ref, fb_ref,
                       pg_ref, pb_ref, w1_ref, b1_ref, w2_ref, b2_ref, o_ref):
    cat = _agg_cat(sd_ref[...], deg_ref[...], rt_ref[...], fw_ref[...],
                   fb_ref[...], pg_ref[...], pb_ref[...])
    col = col_ref[...]
    acc = jnp.zeros((cat.shape[0], EMB), jnp.float32)
    catb = cat.astype(jnp.bfloat16)
    for c in range(NUM_MOD):
        h = jax.nn.relu(jnp.dot(catb, w1_ref[c].astype(jnp.bfloat16),
                                preferred_element_type=jnp.float32) + b1_ref[c])
        z = _bfdot(h, w2_ref[c]) + b2_ref[c]
        acc = jnp.where(col == c, z, acc)
    o_ref[...] = acc


def _post_color(SD, deg, right, color, p, cp):
    grid = N2 // TILE
    return pl.pallas_call(
        _post_color_kernel,
        grid=(grid,),
        in_specs=[
            pl.BlockSpec((TILE, EMB), lambda i: (i, 0)),
            pl.BlockSpec((TILE, 16), lambda i: (i, 0)),
            pl.BlockSpec((TILE, EMB), lambda i: (i, 0)),
            pl.BlockSpec((TILE, 1), lambda i: (i, 0)),
            pl.BlockSpec((EMB, EMB), lambda i: (0, 0)),
            pl.BlockSpec((EMB,), lambda i: (0,)),
            pl.BlockSpec((EMB,), lambda i: (0,)),
            pl.BlockSpec((EMB,), lambda i: (0,)),
            pl.BlockSpec((NUM_MOD, 2 * EMB, EMB), lambda i: (0, 0, 0)),
            pl.BlockSpec((NUM_MOD, EMB), lambda i: (0, 0)),
            pl.BlockSpec((NUM_MOD, EMB, EMB), lambda i: (0, 0, 0)),
            pl.BlockSpec((NUM_MOD, EMB), lambda i: (0, 0)),
        ],
        out_specs=pl.BlockSpec((TILE, EMB), lambda i: (i, 0)),
        out_shape=jax.ShapeDtypeStruct((N2, EMB), jnp.float32),
    )(SD, deg, right, color, p['fmf_W'], p['fmf_b'], p['post_ln_g'],
      p['post_ln_b'], cp['W1'], cp['b1'], cp['W2'], cp['b2'])


def _head_kernel(x_ref, w1_ref, b1_ref, w2_ref, o_ref):
    h = jax.nn.relu(_bfdot(x_ref[...], w1_ref[...]) + b1_ref[...])
    o_ref[...] = _bfdot(h, w2_ref[...])


def _head(x, w1, b1, w2):
    grid = N2 // TILE
    w2p = jnp.pad(w2, ((0, 0), (0, 127)))
    return pl.pallas_call(
        _head_kernel,
        grid=(grid,),
        in_specs=[
            pl.BlockSpec((TILE, EMB), lambda i: (i, 0)),
            pl.BlockSpec((EMB, EMB), lambda i: (0, 0)),
            pl.BlockSpec((EMB,), lambda i: (0,)),
            pl.BlockSpec((EMB, 128), lambda i: (0, 0)),
        ],
        out_specs=pl.BlockSpec((TILE, 128), lambda i: (i, 0)),
        out_shape=jax.ShapeDtypeStruct((N2, 128), jnp.float32),
    )(x, w1, b1, w2p)


def _to_layout(x):
    z = jnp.zeros((NH - H,) + x.shape[1:], x.dtype)
    return jnp.concatenate([x[:H], z, x[H:], z], axis=0)


def kernel(constraint_features, edge_indices, edge_features, variable_features,
           variableColor, consColor, params):
    cons = _emb2(_to_layout(constraint_features), params['cons_emb'])
    var = _emb2(_to_layout(variable_features), params['var_emb'])

    c_vec = params['edge_ln']['b'][0]

    ccol = _to_layout(consColor)[:, None]
    vcol = _to_layout(variableColor)[:, None]

    ec = edge_indices[0]
    ev = edge_indices[1]
    ecp = ec + (ec >= H).astype(jnp.int32) * (NH - H)
    evp = ev + (ev >= H).astype(jnp.int32) * (NH - H)
    npad_e = E2 - ec.shape[0]
    padd = jnp.full((npad_e,), TGD, jnp.int32)
    padz = jnp.zeros((npad_e,), jnp.int32)

    cd_vc, cs_vc, cnt_vc = _sc_partition(jnp.concatenate([ecp, padd]),
                                         jnp.concatenate([evp, padz]))
    cd_cv, cs_cv, cnt_cv = _sc_partition(jnp.concatenate([evp, padd]),
                                         jnp.concatenate([ecp, padz]))
    deg_vc = _sc_deg(cd_vc, cnt_vc)
    deg_cv = _sc_deg(cd_cv, cnt_cv)

    def conv(left, right, part, p, color=None, cp=None):
        cd, cs, cnt, deg = part
        bias = p['fml_b'] + c_vec * p['fme_W'][0]
        A, B = _ab(right, left, p['fml_W'], bias, p['fmr_W'])
        p = dict(p)
        p['fmf_W'] = p['fmf_W'].astype(jnp.bfloat16).astype(jnp.float32)
        SD = _sc_edge(A, B, cd, cs, cnt, p['fmf_ln_g'], p['fmf_ln_b'])
        if color is None:
            return _post_plain(SD, deg, right, p)
        return _post_color(SD, deg, right, color, p, cp)

    cp = params['color']
    part_vc = (cd_vc, cs_vc, cnt_vc, deg_vc)
    part_cv = (cd_cv, cs_cv, cnt_cv, deg_cv)
    cons = conv(var, cons, part_vc, params['conv_v_to_c'], ccol, cp)
    var = conv(cons, var, part_cv, params['conv_c_to_v'], vcol, cp)
    cons = conv(var, cons, part_vc, params['conv_v_to_c2'])
    var = conv(cons, var, part_cv, params['conv_c_to_v2'])

    out = _head(var, params['out']['W1'], params['out']['b1'],
                params['out']['W2'])
    return jnp.concatenate([out[:H, 0], out[NH:NH + H, 0]])
